# baseline (speedup 1.0000x reference)
import functools
import math

import jax
import jax.numpy as jnp
from jax import lax
from jax.experimental import pallas as pl
from jax.experimental.pallas import tpu as pltpu

D_MODEL = 768
NUM_HEADS = 12
HEAD_DIM = 64
D_FF = 3072
S_LEN = 4096
NUM_LOCAL = 614
L_PAD = 640
LBLK = 128
N_LBLK = 5
BF = jnp.bfloat16


def _ln(xb, g, b, eps=1e-5):
    mu = jnp.mean(xb, axis=-1, keepdims=True)
    var = jnp.mean((xb - mu) ** 2, axis=-1, keepdims=True)
    return (xb - mu) * jax.lax.rsqrt(var + eps) * g + b


def _kv_body(x_ref, g_ref, be_ref, wk_ref, bk_ref, wv_ref, bv_ref,
             k_ref, v_ref, wkb, wvb):
    @pl.when(pl.program_id(0) == 0)
    def _():
        wkb[...] = wk_ref[...].astype(BF)
        wvb[...] = wv_ref[...].astype(BF)

    xn = _ln(x_ref[...], g_ref[...], be_ref[...]).astype(BF)
    k_ref[...] = (jnp.dot(xn, wkb[...], preferred_element_type=jnp.float32)
                  + bk_ref[...]).astype(BF)
    v_ref[...] = (jnp.dot(xn, wvb[...], preferred_element_type=jnp.float32)
                  + bv_ref[...]).astype(BF)


def _kv_proj(x, g1, be1, Wk, bk, Wv, bv):
    blk = 512
    return pl.pallas_call(
        _kv_body,
        grid=(S_LEN // blk,),
        in_specs=[
            pl.BlockSpec((blk, D_MODEL), lambda i: (i, 0)),
            pl.BlockSpec((D_MODEL,), lambda i: (0,)),
            pl.BlockSpec((D_MODEL,), lambda i: (0,)),
            pl.BlockSpec((D_MODEL, D_MODEL), lambda i: (0, 0)),
            pl.BlockSpec((D_MODEL,), lambda i: (0,)),
            pl.BlockSpec((D_MODEL, D_MODEL), lambda i: (0, 0)),
            pl.BlockSpec((D_MODEL,), lambda i: (0,)),
        ],
        out_specs=[
            pl.BlockSpec((blk, D_MODEL), lambda i: (i, 0)),
            pl.BlockSpec((blk, D_MODEL), lambda i: (i, 0)),
        ],
        out_shape=[
            jax.ShapeDtypeStruct((S_LEN, D_MODEL), BF),
            jax.ShapeDtypeStruct((S_LEN, D_MODEL), BF),
        ],
        scratch_shapes=[pltpu.VMEM((D_MODEL, D_MODEL), BF)] * 2,
    )(x, g1, be1, Wk, bk, Wv, bv)


def _topk_body(vcol_ref, vrow_ref, idx_ref):
    ii = lax.broadcasted_iota(jnp.int32, (S_LEN, 1), 0)
    vcol = vcol_ref[...]

    cblk = 1024
    nchunk = S_LEN // cblk

    def rank_step(c, cnt):
        vj = vrow_ref[:, pl.ds(c * cblk, cblk)]
        jj = c * cblk + lax.broadcasted_iota(jnp.int32, (1, cblk), 1)
        beats = (vj > vcol) | ((vj == vcol) & (jj < ii))
        return cnt + jnp.sum(beats.astype(jnp.float32), axis=1, keepdims=True)

    rank = lax.fori_loop(0, nchunk, rank_step, jnp.zeros((S_LEN, 1), jnp.float32))

    ii_f = ii.astype(jnp.float32)
    for rc in range(L_PAD // 128):
        rr = (rc * 128 + lax.broadcasted_iota(jnp.int32, (1, 128), 1)).astype(jnp.float32)
        eq = (rank == rr).astype(jnp.float32)
        idx = jnp.sum(eq * ii_f, axis=0, keepdims=True)
        idx_ref[:, pl.ds(rc * 128, 128)] = idx.astype(jnp.int32)


def _topk(cls_col, cls_row):
    return pl.pallas_call(
        _topk_body,
        in_specs=[
            pl.BlockSpec(memory_space=pltpu.VMEM),
            pl.BlockSpec(memory_space=pltpu.VMEM),
        ],
        out_specs=pl.BlockSpec(memory_space=pltpu.VMEM),
        out_shape=jax.ShapeDtypeStruct((1, L_PAD), jnp.int32),
    )(cls_col, cls_row)


def _q_body(idx_ref, x_ref, g_ref, be_ref, wq_ref, bq_ref, q_ref, gat, wqb):
    lb = pl.program_id(0)

    @pl.when(lb == 0)
    def _():
        wqb[...] = wq_ref[...].astype(BF)

    def gather_step(i, _):
        gat[pl.ds(i, 1), :] = x_ref[pl.ds(idx_ref[0, lb * LBLK + i], 1), :]
        return 0

    lax.fori_loop(0, LBLK, gather_step, 0)
    xn = _ln(gat[...], g_ref[...], be_ref[...]).astype(BF)
    scale = 1.0 / math.sqrt(HEAD_DIM)
    q_ref[...] = ((jnp.dot(xn, wqb[...], preferred_element_type=jnp.float32)
                   + bq_ref[...]) * scale).astype(BF)


def _q_proj(top_idx, x, g1, be1, Wq, bq):
    return pl.pallas_call(
        _q_body,
        grid=(N_LBLK,),
        in_specs=[
            pl.BlockSpec(memory_space=pltpu.SMEM),
            pl.BlockSpec((S_LEN, D_MODEL), lambda i: (0, 0)),
            pl.BlockSpec((D_MODEL,), lambda i: (0,)),
            pl.BlockSpec((D_MODEL,), lambda i: (0,)),
            pl.BlockSpec((D_MODEL, D_MODEL), lambda i: (0, 0)),
            pl.BlockSpec((D_MODEL,), lambda i: (0,)),
        ],
        out_specs=pl.BlockSpec((LBLK, D_MODEL), lambda i: (i, 0)),
        out_shape=jax.ShapeDtypeStruct((L_PAD, D_MODEL), BF),
        scratch_shapes=[pltpu.VMEM((LBLK, D_MODEL), jnp.float32),
                        pltpu.VMEM((D_MODEL, D_MODEL), BF)],
    )(top_idx, x, g1, be1, Wq, bq)


ABLK = 160
HGRP = 6
HCOL = HGRP * HEAD_DIM


def _attn_body(q_ref, k_ref, v_ref, wo_ref, probs_ref, lo_ref, wob):
    @pl.when(pl.program_id(0) == 0)
    def _():
        wob[...] = wo_ref[...].astype(BF)

    aos = []
    for h in range(HGRP):
        qh = q_ref[:, h * HEAD_DIM:(h + 1) * HEAD_DIM]
        kh = k_ref[:, h * HEAD_DIM:(h + 1) * HEAD_DIM]
        vh = v_ref[:, h * HEAD_DIM:(h + 1) * HEAD_DIM]
        s = lax.dot_general(qh, kh, (((1,), (1,)), ((), ())),
                            preferred_element_type=jnp.float32)
        m = jnp.max(s, axis=1, keepdims=True)
        p = jnp.exp(s - m)
        inv = 1.0 / jnp.sum(p, axis=1, keepdims=True)
        probs = p * inv
        probs_ref[h] = probs
        aos.append(jnp.dot(probs.astype(BF), vh,
                           preferred_element_type=jnp.float32))
    acc = jnp.concatenate(aos, axis=1)
    lo_ref[...] = jnp.dot(acc.astype(BF), wob[...],
                          preferred_element_type=jnp.float32)


def _attention_half(q, k, v, Wo, hg):
    return pl.pallas_call(
        _attn_body,
        grid=(L_PAD // ABLK,),
        in_specs=[
            pl.BlockSpec((ABLK, HCOL), lambda l: (l, hg)),
            pl.BlockSpec((S_LEN, HCOL), lambda l: (0, hg)),
            pl.BlockSpec((S_LEN, HCOL), lambda l: (0, hg)),
            pl.BlockSpec((HCOL, D_MODEL), lambda l: (hg, 0)),
        ],
        out_specs=[
            pl.BlockSpec((HGRP, ABLK, S_LEN), lambda l: (0, l, 0)),
            pl.BlockSpec((ABLK, D_MODEL), lambda l: (l, 0)),
        ],
        out_shape=[
            jax.ShapeDtypeStruct((HGRP, NUM_LOCAL, S_LEN), jnp.float32),
            jax.ShapeDtypeStruct((L_PAD, D_MODEL), jnp.float32),
        ],
        scratch_shapes=[pltpu.VMEM((HCOL, D_MODEL), BF)],
    )(q, k, v, Wo)


def _scatter_body(idx_ref, alpha_ref, x_ref, lo1_ref, lo2_ref, bo_ref, out_ref):
    out_ref[...] = x_ref[...]
    alpha = alpha_ref[0]

    def step(i, _):
        idx = idx_ref[0, i]
        lo = lo1_ref[pl.ds(i, 1), :] + lo2_ref[pl.ds(i, 1), :] + bo_ref[...]
        out_ref[pl.ds(idx, 1), :] = out_ref[pl.ds(idx, 1), :] + alpha * lo
        return 0

    lax.fori_loop(0, NUM_LOCAL, step, 0)


def _scatter(top_idx, alpha, x, lo1, lo2, bo):
    return pl.pallas_call(
        _scatter_body,
        in_specs=[
            pl.BlockSpec(memory_space=pltpu.SMEM),
            pl.BlockSpec(memory_space=pltpu.SMEM),
            pl.BlockSpec(memory_space=pltpu.VMEM),
            pl.BlockSpec(memory_space=pltpu.VMEM),
            pl.BlockSpec(memory_space=pltpu.VMEM),
            pl.BlockSpec((1, D_MODEL), memory_space=pltpu.VMEM),
        ],
        out_specs=pl.BlockSpec(memory_space=pltpu.VMEM),
        out_shape=jax.ShapeDtypeStruct((S_LEN, D_MODEL), jnp.float32),
    )(top_idx, alpha, x, lo1, lo2, bo)


def _ffn_body(x_ref, g_ref, be_ref, w1_ref, b1_ref, w2_ref, b2_ref, o_ref,
              w1b, w2b):
    @pl.when(pl.program_id(0) == 0)
    def _():
        w1b[...] = w1_ref[...].astype(BF)
        w2b[...] = w2_ref[...].astype(BF)

    xb = x_ref[...]
    xn = _ln(xb, g_ref[...], be_ref[...]).astype(BF)
    h = jnp.dot(xn, w1b[...], preferred_element_type=jnp.float32) + b1_ref[...]
    h = 0.5 * h * (1.0 + lax.erf(h * (1.0 / math.sqrt(2.0))))
    ff = jnp.dot(h.astype(BF), w2b[...],
                 preferred_element_type=jnp.float32) + b2_ref[...]
    o_ref[...] = xb + ff


def _ffn(x_local, g2, be2, W1, b1, W2, b2):
    blk = 512
    return pl.pallas_call(
        _ffn_body,
        grid=(S_LEN // blk,),
        in_specs=[
            pl.BlockSpec((blk, D_MODEL), lambda i: (i, 0)),
            pl.BlockSpec((D_MODEL,), lambda i: (0,)),
            pl.BlockSpec((D_MODEL,), lambda i: (0,)),
            pl.BlockSpec((D_MODEL, D_FF), lambda i: (0, 0)),
            pl.BlockSpec((D_FF,), lambda i: (0,)),
            pl.BlockSpec((D_FF, D_MODEL), lambda i: (0, 0)),
            pl.BlockSpec((D_MODEL,), lambda i: (0,)),
        ],
        out_specs=pl.BlockSpec((blk, D_MODEL), lambda i: (i, 0)),
        out_shape=jax.ShapeDtypeStruct((S_LEN, D_MODEL), jnp.float32),
        scratch_shapes=[pltpu.VMEM((D_MODEL, D_FF), BF),
                        pltpu.VMEM((D_FF, D_MODEL), BF)],
    )(x_local, g2, be2, W1, b1, W2, b2)


def kernel(x, attention_rollout, Wq, bq, Wk, bk, Wv, bv, Wo, bo, W1, b1, W2, b2, g1, be1, g2, be2, alpha):
    B = x.shape[0]
    x2 = x.reshape(S_LEN, D_MODEL)
    cls = attention_rollout[0, 0, :].at[0].set(-1.0)
    cls_col = cls.reshape(S_LEN, 1)
    cls_row = cls.reshape(1, S_LEN)

    k, v = _kv_proj(x2, g1, be1, Wk, bk, Wv, bv)
    top_idx = _topk(cls_col, cls_row)
    q = _q_proj(top_idx, x2, g1, be1, Wq, bq)
    ph0, lo1 = _attention_half(q, k, v, Wo, 0)
    ph1, lo2 = _attention_half(q, k, v, Wo, 1)
    x_local = _scatter(top_idx, alpha, x2, lo1, lo2, bo.reshape(1, D_MODEL))
    x_final = _ffn(x_local, g2, be2, W1, b1, W2, b2)
    probs = jnp.concatenate([ph0, ph1], axis=0)

    return (x_final.reshape(B, S_LEN, D_MODEL),
            probs.reshape(B, NUM_HEADS, NUM_LOCAL, S_LEN))


# speedup vs baseline: 1.3133x; 1.3133x over previous
import functools
import math

import jax
import jax.numpy as jnp
from jax import lax
from jax.experimental import pallas as pl
from jax.experimental.pallas import tpu as pltpu

D_MODEL = 768
NUM_HEADS = 12
HEAD_DIM = 64
D_FF = 3072
S_LEN = 4096
NUM_LOCAL = 614
L_PAD = 640
LBLK = 128
N_LBLK = 5
BF = jnp.bfloat16


def _ln(xb, g, b, eps=1e-5):
    mu = jnp.mean(xb, axis=-1, keepdims=True)
    var = jnp.mean((xb - mu) ** 2, axis=-1, keepdims=True)
    return (xb - mu) * jax.lax.rsqrt(var + eps) * g + b


def _kv_body(x_ref, g_ref, be_ref, wk_ref, bk_ref, wv_ref, bv_ref,
             k_ref, v_ref, wkb, wvb):
    @pl.when(pl.program_id(0) == 0)
    def _():
        wkb[...] = wk_ref[...].astype(BF)
        wvb[...] = wv_ref[...].astype(BF)

    xn = _ln(x_ref[...], g_ref[...], be_ref[...]).astype(BF)
    k_ref[...] = (jnp.dot(xn, wkb[...], preferred_element_type=jnp.float32)
                  + bk_ref[...]).astype(BF)
    v_ref[...] = (jnp.dot(xn, wvb[...], preferred_element_type=jnp.float32)
                  + bv_ref[...]).astype(BF)


def _kv_proj(x, g1, be1, Wk, bk, Wv, bv):
    blk = 512
    return pl.pallas_call(
        _kv_body,
        grid=(S_LEN // blk,),
        in_specs=[
            pl.BlockSpec((blk, D_MODEL), lambda i: (i, 0)),
            pl.BlockSpec((D_MODEL,), lambda i: (0,)),
            pl.BlockSpec((D_MODEL,), lambda i: (0,)),
            pl.BlockSpec((D_MODEL, D_MODEL), lambda i: (0, 0)),
            pl.BlockSpec((D_MODEL,), lambda i: (0,)),
            pl.BlockSpec((D_MODEL, D_MODEL), lambda i: (0, 0)),
            pl.BlockSpec((D_MODEL,), lambda i: (0,)),
        ],
        out_specs=[
            pl.BlockSpec((blk, D_MODEL), lambda i: (i, 0)),
            pl.BlockSpec((blk, D_MODEL), lambda i: (i, 0)),
        ],
        out_shape=[
            jax.ShapeDtypeStruct((S_LEN, D_MODEL), BF),
            jax.ShapeDtypeStruct((S_LEN, D_MODEL), BF),
        ],
        scratch_shapes=[pltpu.VMEM((D_MODEL, D_MODEL), BF)] * 2,
    )(x, g1, be1, Wk, bk, Wv, bv)


def _topk_body(vcol_ref, vrow_ref, idx_ref):
    ii = lax.broadcasted_iota(jnp.int32, (S_LEN, 1), 0)
    vcol = vcol_ref[...]

    cblk = 1024
    nchunk = S_LEN // cblk

    def rank_step(c, cnt):
        vj = vrow_ref[:, pl.ds(c * cblk, cblk)]
        jj = c * cblk + lax.broadcasted_iota(jnp.int32, (1, cblk), 1)
        beats = (vj > vcol) | ((vj == vcol) & (jj < ii))
        return cnt + jnp.sum(beats.astype(jnp.float32), axis=1, keepdims=True)

    rank = lax.fori_loop(0, nchunk, rank_step, jnp.zeros((S_LEN, 1), jnp.float32))

    ii_f = ii.astype(jnp.float32)
    for rc in range(L_PAD // 128):
        rr = (rc * 128 + lax.broadcasted_iota(jnp.int32, (1, 128), 1)).astype(jnp.float32)
        eq = (rank == rr).astype(jnp.float32)
        idx = jnp.sum(eq * ii_f, axis=0, keepdims=True)
        idx_ref[:, pl.ds(rc * 128, 128)] = idx.astype(jnp.int32)


def _topk(cls_col, cls_row):
    return pl.pallas_call(
        _topk_body,
        in_specs=[
            pl.BlockSpec(memory_space=pltpu.VMEM),
            pl.BlockSpec(memory_space=pltpu.VMEM),
        ],
        out_specs=pl.BlockSpec(memory_space=pltpu.VMEM),
        out_shape=jax.ShapeDtypeStruct((1, L_PAD), jnp.int32),
    )(cls_col, cls_row)


def _q_body(idx_ref, x_ref, g_ref, be_ref, wq_ref, bq_ref, q_ref, gat, wqb):
    lb = pl.program_id(0)

    @pl.when(lb == 0)
    def _():
        wqb[...] = wq_ref[...].astype(BF)

    def gather_step(i, _):
        gat[pl.ds(i, 1), :] = x_ref[pl.ds(idx_ref[0, lb * LBLK + i], 1), :]
        return 0

    lax.fori_loop(0, LBLK, gather_step, 0)
    xn = _ln(gat[...], g_ref[...], be_ref[...]).astype(BF)
    scale = 1.0 / math.sqrt(HEAD_DIM)
    q_ref[...] = ((jnp.dot(xn, wqb[...], preferred_element_type=jnp.float32)
                   + bq_ref[...]) * scale).astype(BF)


def _q_proj(top_idx, x, g1, be1, Wq, bq):
    return pl.pallas_call(
        _q_body,
        grid=(N_LBLK,),
        in_specs=[
            pl.BlockSpec(memory_space=pltpu.SMEM),
            pl.BlockSpec((S_LEN, D_MODEL), lambda i: (0, 0)),
            pl.BlockSpec((D_MODEL,), lambda i: (0,)),
            pl.BlockSpec((D_MODEL,), lambda i: (0,)),
            pl.BlockSpec((D_MODEL, D_MODEL), lambda i: (0, 0)),
            pl.BlockSpec((D_MODEL,), lambda i: (0,)),
        ],
        out_specs=pl.BlockSpec((LBLK, D_MODEL), lambda i: (i, 0)),
        out_shape=jax.ShapeDtypeStruct((L_PAD, D_MODEL), BF),
        scratch_shapes=[pltpu.VMEM((LBLK, D_MODEL), jnp.float32),
                        pltpu.VMEM((D_MODEL, D_MODEL), BF)],
    )(top_idx, x, g1, be1, Wq, bq)


ABLK = 160
HGRP = 6
HCOL = HGRP * HEAD_DIM


def _attn_body(q_ref, k_ref, v_ref, wo_ref, probs_ref, lo_ref):
    hg = pl.program_id(1)
    aos = []
    for h in range(HGRP):
        qh = q_ref[:, h * HEAD_DIM:(h + 1) * HEAD_DIM]
        kh = k_ref[:, h * HEAD_DIM:(h + 1) * HEAD_DIM]
        vh = v_ref[:, h * HEAD_DIM:(h + 1) * HEAD_DIM]
        s = lax.dot_general(qh, kh, (((1,), (1,)), ((), ())),
                            preferred_element_type=jnp.float32)
        m = jnp.max(s, axis=1, keepdims=True)
        p = jnp.exp(s - m)
        inv = 1.0 / jnp.sum(p, axis=1, keepdims=True)
        probs = p * inv
        probs_ref[h] = probs
        aos.append(jnp.dot(probs.astype(BF), vh,
                           preferred_element_type=jnp.float32))
    acc = jnp.concatenate(aos, axis=1)
    partial = jnp.dot(acc.astype(BF), wo_ref[...].astype(BF),
                      preferred_element_type=jnp.float32)

    @pl.when(hg == 0)
    def _():
        lo_ref[...] = partial

    @pl.when(hg != 0)
    def _():
        lo_ref[...] = lo_ref[...] + partial


def _attention(q, k, v, Wo):
    return pl.pallas_call(
        _attn_body,
        grid=(L_PAD // ABLK, NUM_HEADS // HGRP),
        in_specs=[
            pl.BlockSpec((ABLK, HCOL), lambda l, hg: (l, hg)),
            pl.BlockSpec((S_LEN, HCOL), lambda l, hg: (0, hg)),
            pl.BlockSpec((S_LEN, HCOL), lambda l, hg: (0, hg)),
            pl.BlockSpec((HCOL, D_MODEL), lambda l, hg: (hg, 0)),
        ],
        out_specs=[
            pl.BlockSpec((HGRP, ABLK, S_LEN), lambda l, hg: (hg, l, 0)),
            pl.BlockSpec((ABLK, D_MODEL), lambda l, hg: (l, 0)),
        ],
        out_shape=[
            jax.ShapeDtypeStruct((NUM_HEADS, NUM_LOCAL, S_LEN), jnp.float32),
            jax.ShapeDtypeStruct((L_PAD, D_MODEL), jnp.float32),
        ],
    )(q, k, v, Wo)


def _scatter_body(idx_ref, alpha_ref, x_ref, lo_ref, bo_ref, out_ref):
    out_ref[...] = x_ref[...]
    alpha = alpha_ref[0]

    def step(i, _):
        idx = idx_ref[0, i]
        lo = lo_ref[pl.ds(i, 1), :] + bo_ref[...]
        out_ref[pl.ds(idx, 1), :] = out_ref[pl.ds(idx, 1), :] + alpha * lo
        return 0

    lax.fori_loop(0, NUM_LOCAL, step, 0)


def _scatter(top_idx, alpha, x, lo, bo):
    return pl.pallas_call(
        _scatter_body,
        in_specs=[
            pl.BlockSpec(memory_space=pltpu.SMEM),
            pl.BlockSpec(memory_space=pltpu.SMEM),
            pl.BlockSpec(memory_space=pltpu.VMEM),
            pl.BlockSpec(memory_space=pltpu.VMEM),
            pl.BlockSpec((1, D_MODEL), memory_space=pltpu.VMEM),
        ],
        out_specs=pl.BlockSpec(memory_space=pltpu.VMEM),
        out_shape=jax.ShapeDtypeStruct((S_LEN, D_MODEL), jnp.float32),
    )(top_idx, alpha, x, lo, bo)


def _ffn_body(x_ref, g_ref, be_ref, w1_ref, b1_ref, w2_ref, b2_ref, o_ref,
              w1b, w2b):
    @pl.when(pl.program_id(0) == 0)
    def _():
        w1b[...] = w1_ref[...].astype(BF)
        w2b[...] = w2_ref[...].astype(BF)

    xb = x_ref[...]
    xn = _ln(xb, g_ref[...], be_ref[...]).astype(BF)
    h = jnp.dot(xn, w1b[...], preferred_element_type=jnp.float32) + b1_ref[...]
    h = 0.5 * h * (1.0 + lax.erf(h * (1.0 / math.sqrt(2.0))))
    ff = jnp.dot(h.astype(BF), w2b[...],
                 preferred_element_type=jnp.float32) + b2_ref[...]
    o_ref[...] = xb + ff


def _ffn(x_local, g2, be2, W1, b1, W2, b2):
    blk = 512
    return pl.pallas_call(
        _ffn_body,
        grid=(S_LEN // blk,),
        in_specs=[
            pl.BlockSpec((blk, D_MODEL), lambda i: (i, 0)),
            pl.BlockSpec((D_MODEL,), lambda i: (0,)),
            pl.BlockSpec((D_MODEL,), lambda i: (0,)),
            pl.BlockSpec((D_MODEL, D_FF), lambda i: (0, 0)),
            pl.BlockSpec((D_FF,), lambda i: (0,)),
            pl.BlockSpec((D_FF, D_MODEL), lambda i: (0, 0)),
            pl.BlockSpec((D_MODEL,), lambda i: (0,)),
        ],
        out_specs=pl.BlockSpec((blk, D_MODEL), lambda i: (i, 0)),
        out_shape=jax.ShapeDtypeStruct((S_LEN, D_MODEL), jnp.float32),
        scratch_shapes=[pltpu.VMEM((D_MODEL, D_FF), BF),
                        pltpu.VMEM((D_FF, D_MODEL), BF)],
    )(x_local, g2, be2, W1, b1, W2, b2)


def kernel(x, attention_rollout, Wq, bq, Wk, bk, Wv, bv, Wo, bo, W1, b1, W2, b2, g1, be1, g2, be2, alpha):
    B = x.shape[0]
    x2 = x.reshape(S_LEN, D_MODEL)
    cls = attention_rollout[0, 0, :].at[0].set(-1.0)
    cls_col = cls.reshape(S_LEN, 1)
    cls_row = cls.reshape(1, S_LEN)

    k, v = _kv_proj(x2, g1, be1, Wk, bk, Wv, bv)
    top_idx = _topk(cls_col, cls_row)
    q = _q_proj(top_idx, x2, g1, be1, Wq, bq)
    probs, lo = _attention(q, k, v, Wo)
    x_local = _scatter(top_idx, alpha, x2, lo, bo.reshape(1, D_MODEL))
    x_final = _ffn(x_local, g2, be2, W1, b1, W2, b2)

    return (x_final.reshape(B, S_LEN, D_MODEL),
            probs.reshape(B, NUM_HEADS, NUM_LOCAL, S_LEN))


# speedup vs baseline: 1.3498x; 1.0278x over previous
import functools
import math

import jax
import jax.numpy as jnp
from jax import lax
from jax.experimental import pallas as pl
from jax.experimental.pallas import tpu as pltpu
from jax.experimental.pallas import tpu_sc as plsc

D_MODEL = 768
NUM_HEADS = 12
HEAD_DIM = 64
D_FF = 3072
S_LEN = 4096
NUM_LOCAL = 614
L_PAD = 640
IDX_PAD = 768
LBLK = 128
N_LBLK = 5
BF = jnp.bfloat16


def _ln(xb, g, b, eps=1e-5):
    mu = jnp.mean(xb, axis=-1, keepdims=True)
    var = jnp.mean((xb - mu) ** 2, axis=-1, keepdims=True)
    return (xb - mu) * jax.lax.rsqrt(var + eps) * g + b


def _kv_body(x_ref, g_ref, be_ref, wk_ref, bk_ref, wv_ref, bv_ref,
             k_ref, v_ref, wkb, wvb):
    @pl.when(pl.program_id(0) == 0)
    def _():
        wkb[...] = wk_ref[...].astype(BF)
        wvb[...] = wv_ref[...].astype(BF)

    xn = _ln(x_ref[...], g_ref[...], be_ref[...]).astype(BF)
    k_ref[...] = (jnp.dot(xn, wkb[...], preferred_element_type=jnp.float32)
                  + bk_ref[...]).astype(BF)
    v_ref[...] = (jnp.dot(xn, wvb[...], preferred_element_type=jnp.float32)
                  + bv_ref[...]).astype(BF)


def _kv_proj(x, g1, be1, Wk, bk, Wv, bv):
    blk = 512
    return pl.pallas_call(
        _kv_body,
        grid=(S_LEN // blk,),
        in_specs=[
            pl.BlockSpec((blk, D_MODEL), lambda i: (i, 0)),
            pl.BlockSpec((D_MODEL,), lambda i: (0,)),
            pl.BlockSpec((D_MODEL,), lambda i: (0,)),
            pl.BlockSpec((D_MODEL, D_MODEL), lambda i: (0, 0)),
            pl.BlockSpec((D_MODEL,), lambda i: (0,)),
            pl.BlockSpec((D_MODEL, D_MODEL), lambda i: (0, 0)),
            pl.BlockSpec((D_MODEL,), lambda i: (0,)),
        ],
        out_specs=[
            pl.BlockSpec((blk, D_MODEL), lambda i: (i, 0)),
            pl.BlockSpec((blk, D_MODEL), lambda i: (i, 0)),
        ],
        out_shape=[
            jax.ShapeDtypeStruct((S_LEN, D_MODEL), BF),
            jax.ShapeDtypeStruct((S_LEN, D_MODEL), BF),
        ],
        scratch_shapes=[pltpu.VMEM((D_MODEL, D_MODEL), BF)] * 2,
    )(x, g1, be1, Wk, bk, Wv, bv)


def _topk_body(vcol_ref, vrow_ref, idx_ref):
    ii = lax.broadcasted_iota(jnp.int32, (S_LEN, 1), 0)
    vcol = vcol_ref[...]

    cblk = 1024
    nchunk = S_LEN // cblk

    def rank_step(c, cnt):
        vj = vrow_ref[:, pl.ds(c * cblk, cblk)]
        jj = c * cblk + lax.broadcasted_iota(jnp.int32, (1, cblk), 1)
        beats = (vj > vcol) | ((vj == vcol) & (jj < ii))
        return cnt + jnp.sum(beats.astype(jnp.float32), axis=1, keepdims=True)

    rank = lax.fori_loop(0, nchunk, rank_step, jnp.zeros((S_LEN, 1), jnp.float32))

    ii_f = ii.astype(jnp.float32)
    for rc in range(IDX_PAD // 128):
        rr = (rc * 128 + lax.broadcasted_iota(jnp.int32, (1, 128), 1)).astype(jnp.float32)
        eq = (rank == rr).astype(jnp.float32)
        idx = jnp.sum(eq * ii_f, axis=0, keepdims=True)
        idx_ref[:, pl.ds(rc * 128, 128)] = idx.astype(jnp.int32)


def _topk(cls_col, cls_row):
    return pl.pallas_call(
        _topk_body,
        in_specs=[
            pl.BlockSpec(memory_space=pltpu.VMEM),
            pl.BlockSpec(memory_space=pltpu.VMEM),
        ],
        out_specs=pl.BlockSpec(memory_space=pltpu.VMEM),
        out_shape=jax.ShapeDtypeStruct((1, IDX_PAD), jnp.int32),
    )(cls_col, cls_row)


SC_NC, SC_NS = 2, 16
SC_NW = SC_NC * SC_NS
SC_BPW = IDX_PAD // SC_NW


def _sc_gather(table, idx):
    mesh = plsc.VectorSubcoreMesh(core_axis_name="c", subcore_axis_name="s")

    @functools.partial(
        pl.kernel, mesh=mesh,
        out_type=jax.ShapeDtypeStruct((IDX_PAD, D_MODEL), jnp.float32),
        scratch_types=[
            pltpu.VMEM((SC_BPW,), jnp.int32),
            pltpu.VMEM((SC_BPW, D_MODEL), jnp.float32),
            pltpu.SemaphoreType.DMA,
        ],
    )
    def k(table_hbm, idx_hbm, out_hbm, idx_v, rows_v, sem):
        wid = lax.axis_index("s") * SC_NC + lax.axis_index("c")
        base = wid * SC_BPW
        pltpu.sync_copy(idx_hbm.at[pl.ds(base, SC_BPW)], idx_v)
        pltpu.async_copy(table_hbm.at[idx_v], rows_v, sem).wait()
        pltpu.sync_copy(rows_v, out_hbm.at[pl.ds(base, SC_BPW)])

    return k(table, idx)


def _q_body(lx_ref, g_ref, be_ref, wq_ref, bq_ref, q_ref, wqb):
    @pl.when(pl.program_id(0) == 0)
    def _():
        wqb[...] = wq_ref[...].astype(BF)

    xn = _ln(lx_ref[...], g_ref[...], be_ref[...]).astype(BF)
    scale = 1.0 / math.sqrt(HEAD_DIM)
    q_ref[...] = ((jnp.dot(xn, wqb[...], preferred_element_type=jnp.float32)
                   + bq_ref[...]) * scale).astype(BF)


def _q_proj(local_x, g1, be1, Wq, bq):
    return pl.pallas_call(
        _q_body,
        grid=(N_LBLK,),
        in_specs=[
            pl.BlockSpec((LBLK, D_MODEL), lambda i: (i, 0)),
            pl.BlockSpec((D_MODEL,), lambda i: (0,)),
            pl.BlockSpec((D_MODEL,), lambda i: (0,)),
            pl.BlockSpec((D_MODEL, D_MODEL), lambda i: (0, 0)),
            pl.BlockSpec((D_MODEL,), lambda i: (0,)),
        ],
        out_specs=pl.BlockSpec((LBLK, D_MODEL), lambda i: (i, 0)),
        out_shape=jax.ShapeDtypeStruct((L_PAD, D_MODEL), BF),
        scratch_shapes=[pltpu.VMEM((D_MODEL, D_MODEL), BF)],
    )(local_x, g1, be1, Wq, bq)


ABLK = 160
HGRP = 6
HCOL = HGRP * HEAD_DIM


def _attn_body(q_ref, k_ref, v_ref, wo_ref, probs_ref, lo_ref):
    hg = pl.program_id(1)
    aos = []
    for h in range(HGRP):
        qh = q_ref[:, h * HEAD_DIM:(h + 1) * HEAD_DIM]
        kh = k_ref[:, h * HEAD_DIM:(h + 1) * HEAD_DIM]
        vh = v_ref[:, h * HEAD_DIM:(h + 1) * HEAD_DIM]
        s = lax.dot_general(qh, kh, (((1,), (1,)), ((), ())),
                            preferred_element_type=jnp.float32)
        m = jnp.max(s, axis=1, keepdims=True)
        p = jnp.exp(s - m)
        inv = 1.0 / jnp.sum(p, axis=1, keepdims=True)
        probs = p * inv
        probs_ref[h] = probs
        aos.append(jnp.dot(probs.astype(BF), vh,
                           preferred_element_type=jnp.float32))
    acc = jnp.concatenate(aos, axis=1)
    partial = jnp.dot(acc.astype(BF), wo_ref[...].astype(BF),
                      preferred_element_type=jnp.float32)

    @pl.when(hg == 0)
    def _():
        lo_ref[...] = partial

    @pl.when(hg != 0)
    def _():
        lo_ref[...] = lo_ref[...] + partial


def _attention(q, k, v, Wo):
    return pl.pallas_call(
        _attn_body,
        grid=(L_PAD // ABLK, NUM_HEADS // HGRP),
        in_specs=[
            pl.BlockSpec((ABLK, HCOL), lambda l, hg: (l, hg)),
            pl.BlockSpec((S_LEN, HCOL), lambda l, hg: (0, hg)),
            pl.BlockSpec((S_LEN, HCOL), lambda l, hg: (0, hg)),
            pl.BlockSpec((HCOL, D_MODEL), lambda l, hg: (hg, 0)),
        ],
        out_specs=[
            pl.BlockSpec((HGRP, ABLK, S_LEN), lambda l, hg: (hg, l, 0)),
            pl.BlockSpec((ABLK, D_MODEL), lambda l, hg: (l, 0)),
        ],
        out_shape=[
            jax.ShapeDtypeStruct((NUM_HEADS, NUM_LOCAL, S_LEN), jnp.float32),
            jax.ShapeDtypeStruct((L_PAD, D_MODEL), jnp.float32),
        ],
    )(q, k, v, Wo)


def _scatter_body(idx_ref, alpha_ref, x_ref, lo_ref, bo_ref, out_ref):
    out_ref[...] = x_ref[...]
    alpha = alpha_ref[0]

    def step(i, _):
        idx = idx_ref[0, i]
        lo = lo_ref[pl.ds(i, 1), :] + bo_ref[...]
        out_ref[pl.ds(idx, 1), :] = out_ref[pl.ds(idx, 1), :] + alpha * lo
        return 0

    lax.fori_loop(0, NUM_LOCAL, step, 0)


def _scatter(top_idx, alpha, x, lo, bo):
    return pl.pallas_call(
        _scatter_body,
        in_specs=[
            pl.BlockSpec(memory_space=pltpu.SMEM),
            pl.BlockSpec(memory_space=pltpu.SMEM),
            pl.BlockSpec(memory_space=pltpu.VMEM),
            pl.BlockSpec(memory_space=pltpu.VMEM),
            pl.BlockSpec((1, D_MODEL), memory_space=pltpu.VMEM),
        ],
        out_specs=pl.BlockSpec(memory_space=pltpu.VMEM),
        out_shape=jax.ShapeDtypeStruct((S_LEN, D_MODEL), jnp.float32),
    )(top_idx, alpha, x, lo, bo)


def _ffn_body(x_ref, g_ref, be_ref, w1_ref, b1_ref, w2_ref, b2_ref, o_ref,
              w1b, w2b):
    @pl.when(pl.program_id(0) == 0)
    def _():
        w1b[...] = w1_ref[...].astype(BF)
        w2b[...] = w2_ref[...].astype(BF)

    xb = x_ref[...]
    xn = _ln(xb, g_ref[...], be_ref[...]).astype(BF)
    h = jnp.dot(xn, w1b[...], preferred_element_type=jnp.float32) + b1_ref[...]
    h = 0.5 * h * (1.0 + lax.erf(h * (1.0 / math.sqrt(2.0))))
    ff = jnp.dot(h.astype(BF), w2b[...],
                 preferred_element_type=jnp.float32) + b2_ref[...]
    o_ref[...] = xb + ff


def _ffn(x_local, g2, be2, W1, b1, W2, b2):
    blk = 512
    return pl.pallas_call(
        _ffn_body,
        grid=(S_LEN // blk,),
        in_specs=[
            pl.BlockSpec((blk, D_MODEL), lambda i: (i, 0)),
            pl.BlockSpec((D_MODEL,), lambda i: (0,)),
            pl.BlockSpec((D_MODEL,), lambda i: (0,)),
            pl.BlockSpec((D_MODEL, D_FF), lambda i: (0, 0)),
            pl.BlockSpec((D_FF,), lambda i: (0,)),
            pl.BlockSpec((D_FF, D_MODEL), lambda i: (0, 0)),
            pl.BlockSpec((D_MODEL,), lambda i: (0,)),
        ],
        out_specs=pl.BlockSpec((blk, D_MODEL), lambda i: (i, 0)),
        out_shape=jax.ShapeDtypeStruct((S_LEN, D_MODEL), jnp.float32),
        scratch_shapes=[pltpu.VMEM((D_MODEL, D_FF), BF),
                        pltpu.VMEM((D_FF, D_MODEL), BF)],
    )(x_local, g2, be2, W1, b1, W2, b2)


def kernel(x, attention_rollout, Wq, bq, Wk, bk, Wv, bv, Wo, bo, W1, b1, W2, b2, g1, be1, g2, be2, alpha):
    B = x.shape[0]
    x2 = x.reshape(S_LEN, D_MODEL)
    cls = attention_rollout[0, 0, :].at[0].set(-1.0)
    cls_col = cls.reshape(S_LEN, 1)
    cls_row = cls.reshape(1, S_LEN)

    top_idx = _topk(cls_col, cls_row)
    local_x = _sc_gather(x2, top_idx.reshape(IDX_PAD))
    k, v = _kv_proj(x2, g1, be1, Wk, bk, Wv, bv)
    q = _q_proj(local_x, g1, be1, Wq, bq)
    probs, lo = _attention(q, k, v, Wo)
    x_local = _scatter(top_idx, alpha, x2, lo, bo.reshape(1, D_MODEL))
    x_final = _ffn(x_local, g2, be2, W1, b1, W2, b2)

    return (x_final.reshape(B, S_LEN, D_MODEL),
            probs.reshape(B, NUM_HEADS, NUM_LOCAL, S_LEN))


# speedup vs baseline: 1.3522x; 1.0017x over previous
import functools
import math

import jax
import jax.numpy as jnp
from jax import lax
from jax.experimental import pallas as pl
from jax.experimental.pallas import tpu as pltpu
from jax.experimental.pallas import tpu_sc as plsc

D_MODEL = 768
NUM_HEADS = 12
HEAD_DIM = 64
D_FF = 3072
S_LEN = 4096
NUM_LOCAL = 614
L_PAD = 640
IDX_PAD = 768
LBLK = 128
N_LBLK = 5
BF = jnp.bfloat16


def _ln(xb, g, b, eps=1e-5):
    mu = jnp.mean(xb, axis=-1, keepdims=True)
    var = jnp.mean((xb - mu) ** 2, axis=-1, keepdims=True)
    return (xb - mu) * jax.lax.rsqrt(var + eps) * g + b


def _kv_body(x_ref, g_ref, be_ref, wk_ref, bk_ref, wv_ref, bv_ref,
             k_ref, v_ref, wkb, wvb):
    @pl.when(pl.program_id(0) == 0)
    def _():
        wkb[...] = wk_ref[...].astype(BF)
        wvb[...] = wv_ref[...].astype(BF)

    xn = _ln(x_ref[...], g_ref[...], be_ref[...]).astype(BF)
    k_ref[...] = (jnp.dot(xn, wkb[...], preferred_element_type=jnp.float32)
                  + bk_ref[...]).astype(BF)
    v_ref[...] = (jnp.dot(xn, wvb[...], preferred_element_type=jnp.float32)
                  + bv_ref[...]).astype(BF)


def _kv_proj(x, g1, be1, Wk, bk, Wv, bv):
    blk = 1024
    return pl.pallas_call(
        _kv_body,
        grid=(S_LEN // blk,),
        in_specs=[
            pl.BlockSpec((blk, D_MODEL), lambda i: (i, 0)),
            pl.BlockSpec((D_MODEL,), lambda i: (0,)),
            pl.BlockSpec((D_MODEL,), lambda i: (0,)),
            pl.BlockSpec((D_MODEL, D_MODEL), lambda i: (0, 0)),
            pl.BlockSpec((D_MODEL,), lambda i: (0,)),
            pl.BlockSpec((D_MODEL, D_MODEL), lambda i: (0, 0)),
            pl.BlockSpec((D_MODEL,), lambda i: (0,)),
        ],
        out_specs=[
            pl.BlockSpec((blk, D_MODEL), lambda i: (i, 0)),
            pl.BlockSpec((blk, D_MODEL), lambda i: (i, 0)),
        ],
        out_shape=[
            jax.ShapeDtypeStruct((S_LEN, D_MODEL), BF),
            jax.ShapeDtypeStruct((S_LEN, D_MODEL), BF),
        ],
        scratch_shapes=[pltpu.VMEM((D_MODEL, D_MODEL), BF)] * 2,
    )(x, g1, be1, Wk, bk, Wv, bv)


def _topk_body(vcol_ref, vrow_ref, idx_ref):
    ii = lax.broadcasted_iota(jnp.int32, (S_LEN, 1), 0)
    vcol = vcol_ref[...]

    cblk = 1024
    nchunk = S_LEN // cblk

    def rank_step(c, cnt):
        vj = vrow_ref[:, pl.ds(c * cblk, cblk)]
        jj = c * cblk + lax.broadcasted_iota(jnp.int32, (1, cblk), 1)
        beats = (vj > vcol) | ((vj == vcol) & (jj < ii))
        return cnt + jnp.sum(beats.astype(jnp.float32), axis=1, keepdims=True)

    rank = lax.fori_loop(0, nchunk, rank_step, jnp.zeros((S_LEN, 1), jnp.float32))

    ii_f = ii.astype(jnp.float32)
    for rc in range(IDX_PAD // 128):
        rr = (rc * 128 + lax.broadcasted_iota(jnp.int32, (1, 128), 1)).astype(jnp.float32)
        eq = (rank == rr).astype(jnp.float32)
        idx = jnp.sum(eq * ii_f, axis=0, keepdims=True)
        idx_ref[:, pl.ds(rc * 128, 128)] = idx.astype(jnp.int32)


def _topk(cls_col, cls_row):
    return pl.pallas_call(
        _topk_body,
        in_specs=[
            pl.BlockSpec(memory_space=pltpu.VMEM),
            pl.BlockSpec(memory_space=pltpu.VMEM),
        ],
        out_specs=pl.BlockSpec(memory_space=pltpu.VMEM),
        out_shape=jax.ShapeDtypeStruct((1, IDX_PAD), jnp.int32),
    )(cls_col, cls_row)


SC_NC, SC_NS = 2, 16
SC_NW = SC_NC * SC_NS
SC_BPW = IDX_PAD // SC_NW


def _sc_gather(table, idx):
    mesh = plsc.VectorSubcoreMesh(core_axis_name="c", subcore_axis_name="s")

    @functools.partial(
        pl.kernel, mesh=mesh,
        out_type=jax.ShapeDtypeStruct((IDX_PAD, D_MODEL), jnp.float32),
        scratch_types=[
            pltpu.VMEM((SC_BPW,), jnp.int32),
            pltpu.VMEM((SC_BPW, D_MODEL), jnp.float32),
            pltpu.SemaphoreType.DMA,
        ],
    )
    def k(table_hbm, idx_hbm, out_hbm, idx_v, rows_v, sem):
        wid = lax.axis_index("s") * SC_NC + lax.axis_index("c")
        base = wid * SC_BPW
        pltpu.sync_copy(idx_hbm.at[pl.ds(base, SC_BPW)], idx_v)
        pltpu.async_copy(table_hbm.at[idx_v], rows_v, sem).wait()
        pltpu.sync_copy(rows_v, out_hbm.at[pl.ds(base, SC_BPW)])

    return k(table, idx)


def _q_body(lx_ref, g_ref, be_ref, wq_ref, bq_ref, q_ref, wqb):
    @pl.when(pl.program_id(0) == 0)
    def _():
        wqb[...] = wq_ref[...].astype(BF)

    xn = _ln(lx_ref[...], g_ref[...], be_ref[...]).astype(BF)
    scale = 1.0 / math.sqrt(HEAD_DIM)
    q_ref[...] = ((jnp.dot(xn, wqb[...], preferred_element_type=jnp.float32)
                   + bq_ref[...]) * scale).astype(BF)


def _q_proj(local_x, g1, be1, Wq, bq):
    return pl.pallas_call(
        _q_body,
        grid=(N_LBLK,),
        in_specs=[
            pl.BlockSpec((LBLK, D_MODEL), lambda i: (i, 0)),
            pl.BlockSpec((D_MODEL,), lambda i: (0,)),
            pl.BlockSpec((D_MODEL,), lambda i: (0,)),
            pl.BlockSpec((D_MODEL, D_MODEL), lambda i: (0, 0)),
            pl.BlockSpec((D_MODEL,), lambda i: (0,)),
        ],
        out_specs=pl.BlockSpec((LBLK, D_MODEL), lambda i: (i, 0)),
        out_shape=jax.ShapeDtypeStruct((L_PAD, D_MODEL), BF),
        scratch_shapes=[pltpu.VMEM((D_MODEL, D_MODEL), BF)],
    )(local_x, g1, be1, Wq, bq)


ABLK = 160
HGRP = 6
HCOL = HGRP * HEAD_DIM


def _attn_body(q_ref, k_ref, v_ref, wo_ref, probs_ref, lo_ref):
    hg = pl.program_id(1)
    aos = []
    for h in range(HGRP):
        qh = q_ref[:, h * HEAD_DIM:(h + 1) * HEAD_DIM]
        kh = k_ref[:, h * HEAD_DIM:(h + 1) * HEAD_DIM]
        vh = v_ref[:, h * HEAD_DIM:(h + 1) * HEAD_DIM]
        s = lax.dot_general(qh, kh, (((1,), (1,)), ((), ())),
                            preferred_element_type=jnp.float32)
        m = jnp.max(s, axis=1, keepdims=True)
        p = jnp.exp(s - m)
        inv = 1.0 / jnp.sum(p, axis=1, keepdims=True)
        probs = p * inv
        probs_ref[h] = probs
        aos.append(jnp.dot(probs.astype(BF), vh,
                           preferred_element_type=jnp.float32))
    acc = jnp.concatenate(aos, axis=1)
    partial = jnp.dot(acc.astype(BF), wo_ref[...].astype(BF),
                      preferred_element_type=jnp.float32)

    @pl.when(hg == 0)
    def _():
        lo_ref[...] = partial

    @pl.when(hg != 0)
    def _():
        lo_ref[...] = lo_ref[...] + partial


def _attention(q, k, v, Wo):
    return pl.pallas_call(
        _attn_body,
        grid=(L_PAD // ABLK, NUM_HEADS // HGRP),
        in_specs=[
            pl.BlockSpec((ABLK, HCOL), lambda l, hg: (l, hg)),
            pl.BlockSpec((S_LEN, HCOL), lambda l, hg: (0, hg)),
            pl.BlockSpec((S_LEN, HCOL), lambda l, hg: (0, hg)),
            pl.BlockSpec((HCOL, D_MODEL), lambda l, hg: (hg, 0)),
        ],
        out_specs=[
            pl.BlockSpec((HGRP, ABLK, S_LEN), lambda l, hg: (hg, l, 0)),
            pl.BlockSpec((ABLK, D_MODEL), lambda l, hg: (l, 0)),
        ],
        out_shape=[
            jax.ShapeDtypeStruct((NUM_HEADS, NUM_LOCAL, S_LEN), jnp.float32),
            jax.ShapeDtypeStruct((L_PAD, D_MODEL), jnp.float32),
        ],
    )(q, k, v, Wo)


def _scatter_body(idx_ref, alpha_ref, x_ref, lo_ref, bo_ref, out_ref):
    out_ref[...] = x_ref[...]
    alpha = alpha_ref[0]

    def step(i, _):
        idx = idx_ref[0, i]
        lo = lo_ref[pl.ds(i, 1), :] + bo_ref[...]
        out_ref[pl.ds(idx, 1), :] = out_ref[pl.ds(idx, 1), :] + alpha * lo
        return 0

    lax.fori_loop(0, NUM_LOCAL, step, 0)


def _scatter(top_idx, alpha, x, lo, bo):
    return pl.pallas_call(
        _scatter_body,
        in_specs=[
            pl.BlockSpec(memory_space=pltpu.SMEM),
            pl.BlockSpec(memory_space=pltpu.SMEM),
            pl.BlockSpec(memory_space=pltpu.VMEM),
            pl.BlockSpec(memory_space=pltpu.VMEM),
            pl.BlockSpec((1, D_MODEL), memory_space=pltpu.VMEM),
        ],
        out_specs=pl.BlockSpec(memory_space=pltpu.VMEM),
        out_shape=jax.ShapeDtypeStruct((S_LEN, D_MODEL), jnp.float32),
    )(top_idx, alpha, x, lo, bo)


def _ffn_body(x_ref, g_ref, be_ref, w1_ref, b1_ref, w2_ref, b2_ref, o_ref,
              w1b, w2b):
    @pl.when(pl.program_id(0) == 0)
    def _():
        w1b[...] = w1_ref[...].astype(BF)
        w2b[...] = w2_ref[...].astype(BF)

    xb = x_ref[...]
    xn = _ln(xb, g_ref[...], be_ref[...]).astype(BF)
    h = jnp.dot(xn, w1b[...], preferred_element_type=jnp.float32) + b1_ref[...]
    h = 0.5 * h * (1.0 + lax.erf(h * (1.0 / math.sqrt(2.0))))
    ff = jnp.dot(h.astype(BF), w2b[...],
                 preferred_element_type=jnp.float32) + b2_ref[...]
    o_ref[...] = xb + ff


def _ffn(x_local, g2, be2, W1, b1, W2, b2):
    blk = 512
    return pl.pallas_call(
        _ffn_body,
        grid=(S_LEN // blk,),
        in_specs=[
            pl.BlockSpec((blk, D_MODEL), lambda i: (i, 0)),
            pl.BlockSpec((D_MODEL,), lambda i: (0,)),
            pl.BlockSpec((D_MODEL,), lambda i: (0,)),
            pl.BlockSpec((D_MODEL, D_FF), lambda i: (0, 0)),
            pl.BlockSpec((D_FF,), lambda i: (0,)),
            pl.BlockSpec((D_FF, D_MODEL), lambda i: (0, 0)),
            pl.BlockSpec((D_MODEL,), lambda i: (0,)),
        ],
        out_specs=pl.BlockSpec((blk, D_MODEL), lambda i: (i, 0)),
        out_shape=jax.ShapeDtypeStruct((S_LEN, D_MODEL), jnp.float32),
        scratch_shapes=[pltpu.VMEM((D_MODEL, D_FF), BF),
                        pltpu.VMEM((D_FF, D_MODEL), BF)],
    )(x_local, g2, be2, W1, b1, W2, b2)


def kernel(x, attention_rollout, Wq, bq, Wk, bk, Wv, bv, Wo, bo, W1, b1, W2, b2, g1, be1, g2, be2, alpha):
    B = x.shape[0]
    x2 = x.reshape(S_LEN, D_MODEL)
    cls = attention_rollout[0, 0, :].at[0].set(-1.0)
    cls_col = cls.reshape(S_LEN, 1)
    cls_row = cls.reshape(1, S_LEN)

    top_idx = _topk(cls_col, cls_row)
    local_x = _sc_gather(x2, top_idx.reshape(IDX_PAD))
    k, v = _kv_proj(x2, g1, be1, Wk, bk, Wv, bv)
    q = _q_proj(local_x, g1, be1, Wq, bq)
    probs, lo = _attention(q, k, v, Wo)
    x_local = _scatter(top_idx, alpha, x2, lo, bo.reshape(1, D_MODEL))
    x_final = _ffn(x_local, g2, be2, W1, b1, W2, b2)

    return (x_final.reshape(B, S_LEN, D_MODEL),
            probs.reshape(B, NUM_HEADS, NUM_LOCAL, S_LEN))
